# baseline (speedup 1.0000x reference)
import jax
import jax.numpy as jnp
from jax.experimental import pallas as pl

N_KEYS = 100000
BITS = 512
ROWS = 2000
NB = N_KEYS // ROWS


def _body(q_ref, k_ref, o_ref):
    q = q_ref[...]
    k = k_ref[...]
    d = jnp.sum(jnp.abs(k - q), axis=-1)
    o_ref[...] = jnp.where(d <= 1.0, 1.0, 0.0).reshape(1, 1, ROWS)


def kernel(query, keys):
    q = jnp.reshape(query, (1, BITS))
    out = pl.pallas_call(
        _body,
        grid=(NB,),
        in_specs=[
            pl.BlockSpec((1, BITS), lambda i: (0, 0)),
            pl.BlockSpec((ROWS, BITS), lambda i: (i, 0)),
        ],
        out_specs=pl.BlockSpec((1, 1, ROWS), lambda i: (i, 0, 0)),
        out_shape=jax.ShapeDtypeStruct((NB, 1, ROWS), jnp.float32),
    )(q, keys)
    return out.reshape(N_KEYS)


# speedup vs baseline: 1.2498x; 1.2498x over previous
import jax
import jax.numpy as jnp
from jax.experimental import pallas as pl

N_KEYS = 100000
BITS = 512
ROWS = 10000
NB = N_KEYS // ROWS


def _body(q_ref, k_ref, o_ref):
    q = q_ref[...]
    k = k_ref[...]
    d = jnp.sum(jnp.abs(k - q), axis=-1)
    o_ref[...] = jnp.where(d <= 1.0, 1.0, 0.0).reshape(1, 1, ROWS)


def kernel(query, keys):
    q = jnp.reshape(query, (1, BITS))
    out = pl.pallas_call(
        _body,
        grid=(NB,),
        in_specs=[
            pl.BlockSpec((1, BITS), lambda i: (0, 0)),
            pl.BlockSpec((ROWS, BITS), lambda i: (i, 0)),
        ],
        out_specs=pl.BlockSpec((1, 1, ROWS), lambda i: (i, 0, 0)),
        out_shape=jax.ShapeDtypeStruct((NB, 1, ROWS), jnp.float32),
    )(q, keys)
    return out.reshape(N_KEYS)
